# Initial kernel scaffold; baseline (speedup 1.0000x reference)
import jax
import jax.numpy as jnp
from jax.experimental import pallas as pl


def kernel(x, edge_indices, edge_weights, batch, pre_params, graph_params, post_params):
    raise NotImplementedError("write your pallas kernel here")


# speedup vs baseline: 1.1182x; 1.1182x over previous
import functools

import jax
import jax.numpy as jnp
from jax.experimental import pallas as pl
from jax.experimental.pallas import tpu as pltpu

N_NODES = 10000
N_EDGES = 320000
D_FEAT = 128
D = 64
N_GRAPHS = 64

_PREC = jax.lax.Precision.HIGHEST


def _mm(a, b):
    return jax.lax.dot_general(a, b, (((1,), (0,)), ((), ())),
                               preferred_element_type=jnp.float32,
                               precision=_PREC)


def _mm_bf16(a, b):
    return jax.lax.dot_general(a.astype(jnp.bfloat16), b.astype(jnp.bfloat16),
                               (((1,), (0,)), ((), ())),
                               preferred_element_type=jnp.float32,
                               precision=jax.lax.Precision.DEFAULT)


def _pre_mlp_body(x_ref, w_refs, o_ref):
    h = x_ref[...]
    for i in range(0, len(w_refs), 2):
        W = w_refs[i][...]
        b = w_refs[i + 1][...]
        h = jnp.maximum(_mm_bf16(h, W) + b, 0.0)
    o_ref[...] = h


def _pre_mlp(x, weights):
    n = len(weights)
    body = lambda *refs: _pre_mlp_body(refs[0], refs[1:n + 1], refs[n + 1])
    return pl.pallas_call(
        body,
        out_shape=jax.ShapeDtypeStruct((N_NODES, D), jnp.float32),
    )(x, *weights)


def _bn(h, g, b):
    m = jnp.mean(h, axis=0, keepdims=True)
    v = jnp.mean((h - m) ** 2, axis=0, keepdims=True)
    return g * (h - m) / jnp.sqrt(v + 1e-5) + b


def _gin_mlp_body(a0_ref, a1_ref, W1_ref, b1_ref, g1_ref, be1_ref,
                  W2_ref, b2_ref, g2_ref, be2_ref, o_ref):
    aggr = a0_ref[...] + a1_ref[...]
    t = _mm_bf16(aggr, W1_ref[...]) + b1_ref[...]
    t = jnp.maximum(_bn(t, g1_ref[...], be1_ref[...]), 0.0)
    u = _mm_bf16(t, W2_ref[...]) + b2_ref[...]
    o_ref[...] = jnp.maximum(_bn(u, g2_ref[...], be2_ref[...]), 0.0)


def _gin_mlp(a0, a1, params):
    return pl.pallas_call(
        _gin_mlp_body,
        out_shape=jax.ShapeDtypeStruct((N_NODES, D), jnp.float32),
    )(a0, a1, *params)


def _final_body(batch_ref, *refs):
    hs = [refs[i][...] for i in range(6)]
    w = refs[6:-1]
    o_ref = refs[-1]
    gid = jax.lax.broadcasted_iota(jnp.int32, (N_GRAPHS, N_NODES), 0)
    onehot = jnp.where(batch_ref[...][None, :] == gid, 1.0, 0.0)
    pooled = [_mm(onehot, h) for h in hs]
    hc = jnp.concatenate(pooled, axis=1)
    for i in range(0, len(w), 2):
        hc = _mm_bf16(hc, w[i][...]) + w[i + 1][...]
        if i + 2 < len(w):
            hc = jnp.maximum(hc, 0.0)
    o_ref[...] = hc


def _final(batch, hs, post_w):
    return pl.pallas_call(
        _final_body,
        out_shape=jax.ShapeDtypeStruct((N_GRAPHS, 128), jnp.float32),
    )(batch, *hs, *post_w)


def _aggregate(x, src, dst, w):
    msg = w[:, None] * jnp.take(x, src, axis=0)
    aggr = jax.ops.segment_sum(msg, dst, num_segments=N_NODES)
    zero = jnp.zeros_like(aggr)
    return aggr, zero


def kernel(x, edge_indices, edge_weights, batch, pre_params, graph_params, post_params):
    src = edge_indices[0]
    dst = edge_indices[1]

    pre_w = []
    for (W1, b1), (W2, b2) in pre_params:
        pre_w += [W1, b1.reshape(1, -1), W2, b2.reshape(1, -1)]
    h = _pre_mlp(x, pre_w)

    hs = []
    for (W1, b1), (g1, be1), (W2, b2), (g2, be2) in graph_params:
        a0, a1 = _aggregate(h, src, dst, edge_weights)
        params = [W1, b1.reshape(1, -1), g1.reshape(1, -1), be1.reshape(1, -1),
                  W2, b2.reshape(1, -1), g2.reshape(1, -1), be2.reshape(1, -1)]
        h = _gin_mlp(a0, a1, params)
        hs.append(h)

    (W1, b1), (W2, b2) = post_params[0]
    (W3, b3), (W4, b4) = post_params[1]
    (W5, b5), (W6, b6) = post_params[2]
    W6p = jnp.pad(W6, ((0, 0), (0, 127)))
    b6p = jnp.pad(b6, ((0, 127)))
    post_w = [W1, b1.reshape(1, -1), W2, b2.reshape(1, -1),
              W3, b3.reshape(1, -1), W4, b4.reshape(1, -1),
              W5, b5.reshape(1, -1), W6p, b6p.reshape(1, -1)]
    out = _final(batch, hs, post_w)
    return out[:, :1]


# speedup vs baseline: 3.0742x; 2.7492x over previous
import dataclasses
import functools

import jax
import jax.numpy as jnp
from jax import lax
from jax.experimental import pallas as pl
from jax.experimental.pallas import tpu as pltpu
from jax.experimental.pallas import tpu_sc as plsc

N_NODES = 10000
N_EDGES = 320000
D_FEAT = 128
D = 64
N_GRAPHS = 64

_PREC = jax.lax.Precision.HIGHEST


def _mm(a, b):
    return jax.lax.dot_general(a, b, (((1,), (0,)), ((), ())),
                               preferred_element_type=jnp.float32,
                               precision=_PREC)


def _mm_bf16(a, b):
    return jax.lax.dot_general(a.astype(jnp.bfloat16), b.astype(jnp.bfloat16),
                               (((1,), (0,)), ((), ())),
                               preferred_element_type=jnp.float32,
                               precision=jax.lax.Precision.DEFAULT)


def _pre_mlp_body(x_ref, w_refs, o_ref):
    h = x_ref[...]
    for i in range(0, len(w_refs), 2):
        W = w_refs[i][...]
        b = w_refs[i + 1][...]
        h = jnp.maximum(_mm_bf16(h, W) + b, 0.0)
    o_ref[...] = h


def _pre_mlp(x, weights):
    n = len(weights)
    body = lambda *refs: _pre_mlp_body(refs[0], refs[1:n + 1], refs[n + 1])
    return pl.pallas_call(
        body,
        out_shape=jax.ShapeDtypeStruct((N_NODES, D), jnp.float32),
    )(x, *weights)


def _bn(h, g, b):
    m = jnp.mean(h, axis=0, keepdims=True)
    v = jnp.mean((h - m) ** 2, axis=0, keepdims=True)
    return g * (h - m) / jnp.sqrt(v + 1e-5) + b


def _gin_mlp_body(a0_ref, a1_ref, W1_ref, b1_ref, g1_ref, be1_ref,
                  W2_ref, b2_ref, g2_ref, be2_ref, o_ref):
    aggr = a0_ref[...] + a1_ref[...]
    t = _mm_bf16(aggr, W1_ref[...]) + b1_ref[...]
    t = jnp.maximum(_bn(t, g1_ref[...], be1_ref[...]), 0.0)
    u = _mm_bf16(t, W2_ref[...]) + b2_ref[...]
    o_ref[...] = jnp.maximum(_bn(u, g2_ref[...], be2_ref[...]), 0.0)


def _gin_mlp(a0, a1, params):
    return pl.pallas_call(
        _gin_mlp_body,
        out_shape=jax.ShapeDtypeStruct((N_NODES, D), jnp.float32),
    )(a0, a1, *params)


def _final_body(batch_ref, *refs):
    hs = [refs[i][...] for i in range(6)]
    w = refs[6:-1]
    o_ref = refs[-1]
    gid = jax.lax.broadcasted_iota(jnp.int32, (N_GRAPHS, N_NODES), 0)
    onehot = jnp.where(batch_ref[...][None, :] == gid, 1.0, 0.0)
    pooled = [_mm(onehot, h) for h in hs]
    hc = jnp.concatenate(pooled, axis=1)
    for i in range(0, len(w), 2):
        hc = _mm_bf16(hc, w[i][...]) + w[i + 1][...]
        if i + 2 < len(w):
            hc = jnp.maximum(hc, 0.0)
    o_ref[...] = hc


def _final(batch, hs, post_w):
    return pl.pallas_call(
        _final_body,
        out_shape=jax.ShapeDtypeStruct((N_GRAPHS, 128), jnp.float32),
    )(batch, *hs, *post_w)


_NC = 2
_NS = 16
_NW = _NC * _NS
_EPW = N_EDGES // _NW
_WIN = 80
_NWIN = _EPW // _WIN
_RPS = 624
_RPS_LAST = N_NODES - (_NS - 1) * _RPS


def _aggr_body(h_hbm, src_hbm, dst_hbm, w_hbm, zero_hbm, out_hbm,
               src_v, dst_v, w_v, rows_v, aggr_sh, sem):
    cid = lax.axis_index("c")
    sid = lax.axis_index("s")
    wid = cid * _NS + sid

    @pl.when(sid < _NS - 1)
    def _():
        pltpu.sync_copy(zero_hbm.at[pl.ds(0, _RPS)],
                        aggr_sh.at[pl.ds(sid * _RPS, _RPS)])

    @pl.when(sid == _NS - 1)
    def _():
        pltpu.sync_copy(zero_hbm, aggr_sh.at[pl.ds((_NS - 1) * _RPS, _RPS_LAST)])

    plsc.subcore_barrier()

    e0 = wid * _EPW

    @pl.loop(0, _NWIN)
    def _win(i):
        base = e0 + i * _WIN
        pltpu.sync_copy(src_hbm.at[pl.ds(base, _WIN)], src_v)
        pltpu.sync_copy(dst_hbm.at[pl.ds(base, _WIN)], dst_v)
        pltpu.sync_copy(w_hbm.at[pl.ds(base, _WIN)], w_v)
        pltpu.async_copy(h_hbm.at[src_v], rows_v, sem).wait()

        @pl.loop(0, _WIN, step=16)
        def _blk(c):
            for eo in range(16):
                wb = plsc.load_gather(w_v, [jnp.full((16,), c + eo, jnp.int32)])
                for k in range(4):
                    sl = pl.ds(16 * k, 16)
                    rows_v[c + eo, sl] = rows_v[c + eo, sl] * wb

        pltpu.sync_copy(rows_v, aggr_sh.at[dst_v], add=True)

    plsc.subcore_barrier()

    @pl.when(sid < _NS - 1)
    def _():
        pltpu.sync_copy(aggr_sh.at[pl.ds(sid * _RPS, _RPS)],
                        out_hbm.at[cid, pl.ds(sid * _RPS, _RPS)])

    @pl.when(sid == _NS - 1)
    def _():
        pltpu.sync_copy(aggr_sh.at[pl.ds((_NS - 1) * _RPS, _RPS_LAST)],
                        out_hbm.at[cid, pl.ds((_NS - 1) * _RPS, _RPS_LAST)])


def _sc_compiler_params():
    cp = pltpu.CompilerParams()
    cp = dataclasses.replace(cp, needs_layout_passes=False,
                             use_tc_tiling_on_sc=False)
    return cp


@jax.jit
def _aggregate_sc(x, src, dst, w, zero):
    kern = pl.kernel(
        _aggr_body,
        compiler_params=_sc_compiler_params(),
        out_type=jax.ShapeDtypeStruct((_NC, N_NODES, D), jnp.float32),
        mesh=plsc.VectorSubcoreMesh(core_axis_name="c", subcore_axis_name="s"),
        scratch_types=[
            pltpu.VMEM((_WIN,), jnp.int32),
            pltpu.VMEM((_WIN,), jnp.int32),
            pltpu.VMEM((_WIN,), jnp.float32),
            pltpu.VMEM((_WIN, D), jnp.float32),
            pltpu.VMEM_SHARED((N_NODES, D), jnp.float32),
            pltpu.SemaphoreType.DMA,
        ],
    )
    return kern(x, src, dst, w, zero)


def _aggregate(x, src, dst, w, zero):
    parts = _aggregate_sc(x, src, dst, w, zero)
    return parts[0], parts[1]


def kernel(x, edge_indices, edge_weights, batch, pre_params, graph_params, post_params):
    src = edge_indices[0]
    dst = edge_indices[1]

    pre_w = []
    for (W1, b1), (W2, b2) in pre_params:
        pre_w += [W1, b1.reshape(1, -1), W2, b2.reshape(1, -1)]
    h = _pre_mlp(x, pre_w)

    zero = jnp.zeros((_RPS_LAST, D), jnp.float32)
    hs = []
    for (W1, b1), (g1, be1), (W2, b2), (g2, be2) in graph_params:
        a0, a1 = _aggregate(h, src, dst, edge_weights, zero)
        params = [W1, b1.reshape(1, -1), g1.reshape(1, -1), be1.reshape(1, -1),
                  W2, b2.reshape(1, -1), g2.reshape(1, -1), be2.reshape(1, -1)]
        h = _gin_mlp(a0, a1, params)
        hs.append(h)

    (W1, b1), (W2, b2) = post_params[0]
    (W3, b3), (W4, b4) = post_params[1]
    (W5, b5), (W6, b6) = post_params[2]
    W6p = jnp.pad(W6, ((0, 0), (0, 127)))
    b6p = jnp.pad(b6, ((0, 127)))
    post_w = [W1, b1.reshape(1, -1), W2, b2.reshape(1, -1),
              W3, b3.reshape(1, -1), W4, b4.reshape(1, -1),
              W5, b5.reshape(1, -1), W6p, b6p.reshape(1, -1)]
    out = _final(batch, hs, post_w)
    return out[:, :1]


# speedup vs baseline: 6.3000x; 2.0493x over previous
import dataclasses
import functools

import jax
import jax.numpy as jnp
from jax import lax
from jax.experimental import pallas as pl
from jax.experimental.pallas import tpu as pltpu
from jax.experimental.pallas import tpu_sc as plsc

N_NODES = 10000
N_EDGES = 320000
D_FEAT = 128
D = 64
N_GRAPHS = 64

_PREC = jax.lax.Precision.HIGHEST


def _mm(a, b):
    return jax.lax.dot_general(a, b, (((1,), (0,)), ((), ())),
                               preferred_element_type=jnp.float32,
                               precision=_PREC)


def _mm_bf16(a, b):
    return jax.lax.dot_general(a.astype(jnp.bfloat16), b.astype(jnp.bfloat16),
                               (((1,), (0,)), ((), ())),
                               preferred_element_type=jnp.float32,
                               precision=jax.lax.Precision.DEFAULT)


def _pre_mlp_body(x_ref, w_refs, o_ref):
    h = x_ref[...]
    for i in range(0, len(w_refs), 2):
        W = w_refs[i][...]
        b = w_refs[i + 1][...]
        h = jnp.maximum(_mm_bf16(h, W) + b, 0.0)
    o_ref[...] = h


def _pre_mlp(x, weights):
    n = len(weights)
    body = lambda *refs: _pre_mlp_body(refs[0], refs[1:n + 1], refs[n + 1])
    return pl.pallas_call(
        body,
        out_shape=jax.ShapeDtypeStruct((N_NODES, D), jnp.float32),
    )(x, *weights)


def _bn(h, g, b):
    m = jnp.mean(h, axis=0, keepdims=True)
    v = jnp.mean((h - m) ** 2, axis=0, keepdims=True)
    return g * (h - m) / jnp.sqrt(v + 1e-5) + b


def _gin_mlp_body(a0_ref, a1_ref, W1_ref, b1_ref, g1_ref, be1_ref,
                  W2_ref, b2_ref, g2_ref, be2_ref, o_ref):
    aggr = a0_ref[...] + a1_ref[...]
    t = _mm_bf16(aggr, W1_ref[...]) + b1_ref[...]
    t = jnp.maximum(_bn(t, g1_ref[...], be1_ref[...]), 0.0)
    u = _mm_bf16(t, W2_ref[...]) + b2_ref[...]
    o_ref[...] = jnp.maximum(_bn(u, g2_ref[...], be2_ref[...]), 0.0)


def _gin_mlp(a0, a1, params):
    return pl.pallas_call(
        _gin_mlp_body,
        out_shape=jax.ShapeDtypeStruct((N_NODES, D), jnp.float32),
    )(a0, a1, *params)


def _final_body(batch_ref, *refs):
    hs = [refs[i][...] for i in range(6)]
    w = refs[6:-1]
    o_ref = refs[-1]
    gid = jax.lax.broadcasted_iota(jnp.int32, (N_GRAPHS, N_NODES), 0)
    onehot = jnp.where(batch_ref[...][None, :] == gid, 1.0, 0.0)
    pooled = [_mm(onehot, h) for h in hs]
    hc = jnp.concatenate(pooled, axis=1)
    for i in range(0, len(w), 2):
        hc = _mm_bf16(hc, w[i][...]) + w[i + 1][...]
        if i + 2 < len(w):
            hc = jnp.maximum(hc, 0.0)
    o_ref[...] = hc


def _final(batch, hs, post_w):
    return pl.pallas_call(
        _final_body,
        out_shape=jax.ShapeDtypeStruct((N_GRAPHS, 128), jnp.float32),
    )(batch, *hs, *post_w)


_NC = 2
_NS = 16
_NW = _NC * _NS
_EPW = N_EDGES // _NW
_WIN = 100
_NWIN = _EPW // _WIN
_RPS = 624
_RPS_LAST = N_NODES - (_NS - 1) * _RPS


def _aggr_body(h_hbm, src_hbm, dst_hbm, w_hbm, zero_hbm, out_hbm,
               src_all, dst_all, w_all, rin0, rin1, rout0, rout1,
               aggr_sh, gs0, gs1, ss0, ss1):
    cid = lax.axis_index("c")
    sid = lax.axis_index("s")
    wid = cid * _NS + sid

    @pl.when(sid < _NS - 1)
    def _():
        pltpu.sync_copy(zero_hbm.at[pl.ds(0, _RPS)],
                        aggr_sh.at[pl.ds(sid * _RPS, _RPS)])

    @pl.when(sid == _NS - 1)
    def _():
        pltpu.sync_copy(zero_hbm, aggr_sh.at[pl.ds((_NS - 1) * _RPS, _RPS_LAST)])

    r0 = wid * _NWIN
    pltpu.sync_copy(src_hbm.at[pl.ds(r0, _NWIN)], src_all)
    pltpu.sync_copy(dst_hbm.at[pl.ds(r0, _NWIN)], dst_all)
    pltpu.sync_copy(w_hbm.at[pl.ds(r0, _NWIN)], w_all)
    plsc.subcore_barrier()

    rins = (rin0, rin1)
    routs = (rout0, rout1)
    gsems = (gs0, gs1)
    ssems = (ss0, ss1)

    pltpu.async_copy(h_hbm.at[src_all.at[0]], rin0, gs0)
    pltpu.async_copy(h_hbm.at[src_all.at[1]], rin1, gs1)

    @pl.loop(0, _NWIN, step=2)
    def _win(i):
        for b in range(2):
            j = i + b
            rin, rout = rins[b], routs[b]
            pltpu.make_async_copy(h_hbm.at[src_all.at[j]], rin, gsems[b]).wait()

            @pl.when(j >= 2)
            def _():
                pltpu.make_async_copy(
                    rout, aggr_sh.at[dst_all.at[j]], ssems[b]).wait()

            @pl.loop(0, _WIN, step=20)
            def _blk(c):
                for eo in range(20):
                    e = c + eo
                    wb = plsc.load_gather(
                        w_all, [jnp.full((16,), j, jnp.int32),
                                jnp.full((16,), e, jnp.int32)])
                    for k in range(4):
                        sl = pl.ds(16 * k, 16)
                        rout[e, sl] = rin[e, sl] * wb

            @pl.when(j + 2 < _NWIN)
            def _():
                pltpu.async_copy(h_hbm.at[src_all.at[j + 2]], rin, gsems[b])

            pltpu.async_copy(rout, aggr_sh.at[dst_all.at[j]], ssems[b],
                             add=True)

    pltpu.make_async_copy(rout0, aggr_sh.at[dst_all.at[_NWIN - 2]], ss0).wait()
    pltpu.make_async_copy(rout1, aggr_sh.at[dst_all.at[_NWIN - 1]], ss1).wait()
    plsc.subcore_barrier()

    @pl.when(sid < _NS - 1)
    def _():
        pltpu.sync_copy(aggr_sh.at[pl.ds(sid * _RPS, _RPS)],
                        out_hbm.at[cid, pl.ds(sid * _RPS, _RPS)])

    @pl.when(sid == _NS - 1)
    def _():
        pltpu.sync_copy(aggr_sh.at[pl.ds((_NS - 1) * _RPS, _RPS_LAST)],
                        out_hbm.at[cid, pl.ds((_NS - 1) * _RPS, _RPS_LAST)])


def _sc_compiler_params():
    cp = pltpu.CompilerParams()
    cp = dataclasses.replace(cp, needs_layout_passes=False,
                             use_tc_tiling_on_sc=False)
    return cp


@jax.jit
def _aggregate_sc(x, src, dst, w, zero):
    kern = pl.kernel(
        _aggr_body,
        compiler_params=_sc_compiler_params(),
        out_type=jax.ShapeDtypeStruct((_NC, N_NODES, D), jnp.float32),
        mesh=plsc.VectorSubcoreMesh(core_axis_name="c", subcore_axis_name="s"),
        scratch_types=[
            pltpu.VMEM((_NWIN, _WIN), jnp.int32),
            pltpu.VMEM((_NWIN, _WIN), jnp.int32),
            pltpu.VMEM((_NWIN, _WIN), jnp.float32),
            pltpu.VMEM((_WIN, D), jnp.float32),
            pltpu.VMEM((_WIN, D), jnp.float32),
            pltpu.VMEM((_WIN, D), jnp.float32),
            pltpu.VMEM((_WIN, D), jnp.float32),
            pltpu.VMEM_SHARED((N_NODES, D), jnp.float32),
            pltpu.SemaphoreType.DMA,
            pltpu.SemaphoreType.DMA,
            pltpu.SemaphoreType.DMA,
            pltpu.SemaphoreType.DMA,
        ],
    )
    return kern(x, src.reshape(_NW * _NWIN, _WIN), dst.reshape(_NW * _NWIN, _WIN),
                w.reshape(_NW * _NWIN, _WIN), zero)


def _aggregate(x, src, dst, w, zero):
    parts = _aggregate_sc(x, src, dst, w, zero)
    return parts[0], parts[1]


def kernel(x, edge_indices, edge_weights, batch, pre_params, graph_params, post_params):
    src = edge_indices[0]
    dst = edge_indices[1]

    pre_w = []
    for (W1, b1), (W2, b2) in pre_params:
        pre_w += [W1, b1.reshape(1, -1), W2, b2.reshape(1, -1)]
    h = _pre_mlp(x, pre_w)

    zero = jnp.zeros((_RPS_LAST, D), jnp.float32)
    hs = []
    for (W1, b1), (g1, be1), (W2, b2), (g2, be2) in graph_params:
        a0, a1 = _aggregate(h, src, dst, edge_weights, zero)
        params = [W1, b1.reshape(1, -1), g1.reshape(1, -1), be1.reshape(1, -1),
                  W2, b2.reshape(1, -1), g2.reshape(1, -1), be2.reshape(1, -1)]
        h = _gin_mlp(a0, a1, params)
        hs.append(h)

    (W1, b1), (W2, b2) = post_params[0]
    (W3, b3), (W4, b4) = post_params[1]
    (W5, b5), (W6, b6) = post_params[2]
    W6p = jnp.pad(W6, ((0, 0), (0, 127)))
    b6p = jnp.pad(b6, ((0, 127)))
    post_w = [W1, b1.reshape(1, -1), W2, b2.reshape(1, -1),
              W3, b3.reshape(1, -1), W4, b4.reshape(1, -1),
              W5, b5.reshape(1, -1), W6p, b6p.reshape(1, -1)]
    out = _final(batch, hs, post_w)
    return out[:, :1]
